# baseline (speedup 1.0000x reference)
import functools
import math

import jax
import jax.numpy as jnp
from jax import lax
from jax.experimental import pallas as pl
from jax.experimental.pallas import tpu as pltpu, tpu_sc as plsc

VOCAB_DIM = 64
G_MIN = 0.1

N_ROWS = 4096 * 200
C = 256
SUB = 128
NSUB = C // SUB

_GATHER_DNUMS = lax.GatherDimensionNumbers(
    offset_dims=(), collapsed_slice_dims=(0,), start_index_map=(0,))


def _bcast_lane(v16, r):
    idx = jnp.full((16, 1), r, jnp.int32)
    return lax.gather(v16, idx, _GATHER_DNUMS, (1,),
                      mode=lax.GatherScatterMode.PROMISE_IN_BOUNDS)


def _build_sc_call():
    info = plsc.get_sparse_core_info()
    nc, ns = info.num_cores, info.num_subcores
    nw = nc * ns
    rows_per_w = N_ROWS // nw
    nchunks = rows_per_w // C
    npairs = nchunks // 2

    mesh = plsc.VectorSubcoreMesh(core_axis_name="c", subcore_axis_name="s")

    @functools.partial(
        pl.kernel,
        mesh=mesh,
        compiler_params=pltpu.CompilerParams(use_tc_tiling_on_sc=False),
        out_type=jax.ShapeDtypeStruct((N_ROWS, VOCAB_DIM), jnp.float32),
        scratch_types=[
            pltpu.VMEM((C,), jnp.int32),
            pltpu.VMEM((C,), jnp.int32),
            pltpu.VMEM((C,), jnp.float32),
            pltpu.VMEM((C,), jnp.float32),
            pltpu.VMEM((C, VOCAB_DIM), jnp.float32),
            pltpu.VMEM((C, VOCAB_DIM), jnp.float32),
            pltpu.VMEM((C, VOCAB_DIM), jnp.float32),
            pltpu.VMEM((C, VOCAB_DIM), jnp.float32),
            pltpu.SemaphoreType.DMA,
            pltpu.SemaphoreType.DMA,
            pltpu.SemaphoreType.DMA,
            pltpu.SemaphoreType.DMA,
        ],
    )
    def sc_call(ids_h, base_h, prior_h, gate_h, out_h,
                idx0, idx1, gte0, gte1, bb0, bb1, pb0, pb1,
                gsem0, gsem1, ssem0, ssem1):
        wid = lax.axis_index("s") * nc + lax.axis_index("c")
        wbase = wid * rows_per_w
        slots = ((idx0, gte0, bb0, pb0, gsem0, ssem0),
                 (idx1, gte1, bb1, pb1, gsem1, ssem1))

        def fire_gathers(c, slot):
            idxb, gteb, bb, pb, gsem, _ = slot
            row0 = wbase + c * C
            pltpu.sync_copy(ids_h.at[pl.ds(row0, C)], idxb)
            for j in range(NSUB):
                sl = pl.ds(j * SUB, SUB)
                pltpu.async_copy(base_h.at[idxb.at[sl]], bb.at[sl, :], gsem)
                pltpu.async_copy(prior_h.at[idxb.at[sl]], pb.at[sl, :], gsem)
                pltpu.async_copy(gate_h.at[idxb.at[sl]], gteb.at[sl], gsem)

        def wait_gathers(slot):
            idxb, gteb, bb, pb, gsem, _ = slot
            for j in range(NSUB):
                sl = pl.ds(j * SUB, SUB)
                pltpu.make_async_copy(
                    base_h.at[idxb.at[sl]], bb.at[sl, :], gsem).wait()
                pltpu.make_async_copy(
                    prior_h.at[idxb.at[sl]], pb.at[sl, :], gsem).wait()
                pltpu.make_async_copy(
                    gate_h.at[idxb.at[sl]], gteb.at[sl], gsem).wait()

        def compute(slot):
            _, gteb, bb, pb, _, _ = slot

            def group(i, carry):
                g16 = gteb[pl.ds(i * 16, 16)]
                w16 = G_MIN + (1.0 - G_MIN) / (1.0 + jnp.exp(-g16))
                for r in range(16):
                    row = i * 16 + r
                    wr = _bcast_lane(w16, r)
                    for dc in range(VOCAB_DIM // 16):
                        dsl = pl.ds(dc * 16, 16)
                        bb[row, dsl] = bb[row, dsl] + wr * pb[row, dsl]
                return carry

            lax.fori_loop(0, C // 16, group, 0)

        def fire_store(c, slot):
            _, _, bb, _, _, ssem = slot
            row0 = wbase + c * C
            pltpu.async_copy(bb, out_h.at[pl.ds(row0, C)], ssem)

        def wait_store(c, slot):
            _, _, bb, _, _, ssem = slot
            row0 = wbase + c * C
            pltpu.make_async_copy(bb, out_h.at[pl.ds(row0, C)], ssem).wait()

        fire_gathers(0, slots[0])
        fire_gathers(1, slots[1])

        def pair(p, carry):
            c0 = 2 * p
            c1 = c0 + 1
            wait_gathers(slots[0])
            compute(slots[0])
            fire_store(c0, slots[0])

            @pl.when(p < npairs - 1)
            def _():
                wait_store(c0, slots[0])
                fire_gathers(c0 + 2, slots[0])

            wait_gathers(slots[1])
            compute(slots[1])
            fire_store(c1, slots[1])

            @pl.when(p < npairs - 1)
            def _():
                wait_store(c1, slots[1])
                fire_gathers(c1 + 2, slots[1])

            return carry

        lax.fori_loop(0, npairs, pair, 0)
        wait_store(nchunks - 2, slots[0])
        wait_store(nchunks - 1, slots[1])

    return sc_call


_SC_CALL = _build_sc_call()


@jax.jit
def kernel(input_ids, base_weight, prior_matrix, gate_logits):
    ids_flat = input_ids.reshape(-1).astype(jnp.int32)
    out = _SC_CALL(ids_flat, base_weight, prior_matrix, gate_logits)
    return out.reshape(*input_ids.shape, VOCAB_DIM)
